# Initial kernel scaffold; baseline (speedup 1.0000x reference)
import jax
import jax.numpy as jnp
from jax.experimental import pallas as pl


def kernel(x, router_w, router_b, w_up, b_up, w_down, b_down):
    raise NotImplementedError("write your pallas kernel here")


# speedup vs baseline: 1.0508x; 1.0508x over previous
import functools

import jax
import jax.numpy as jnp
from jax.experimental import pallas as pl
from jax.experimental.pallas import tpu as pltpu

D_MODEL = 768
N_EXP = 8
EXP_DIM = 3072
SEQ = 2048
TS = 256


def _moe_kernel(x_ref, rw_ref, rb_ref, wup_ref, bup_ref, wdn_ref, bdn_ref,
                out_ref, probs_ref):
    e = pl.program_id(0)
    s = pl.program_id(1)
    xs = x_ref[pl.ds(s * TS, TS), :]
    xs_bf = xs.astype(jnp.bfloat16)

    @pl.when(e == 0)
    def _():
        logits = jnp.dot(xs_bf, rw_ref[...].astype(jnp.bfloat16),
                         preferred_element_type=jnp.float32) + rb_ref[...]
        m = jnp.max(logits, axis=-1, keepdims=True)
        ex = jnp.exp(logits - m)
        probs_ref[pl.ds(s * TS, TS), :] = ex / jnp.sum(ex, axis=-1,
                                                       keepdims=True)

    probs = probs_ref[pl.ds(s * TS, TS), :]
    lane = jax.lax.broadcasted_iota(jnp.int32, (TS, N_EXP), 1)
    p_e = jnp.sum(jnp.where(lane == e, probs, 0.0), axis=1, keepdims=True)

    h = jnp.dot(xs_bf, wup_ref[0].astype(jnp.bfloat16),
                preferred_element_type=jnp.float32) + bup_ref[0, 0][None, :]
    h = jax.nn.gelu(h)
    h = (h * p_e).astype(jnp.bfloat16)
    contrib = jnp.dot(h, wdn_ref[0].astype(jnp.bfloat16),
                      preferred_element_type=jnp.float32)

    @pl.when(e == 0)
    def _():
        out_ref[pl.ds(s * TS, TS), :] = contrib + jnp.dot(
            probs.astype(jnp.bfloat16), bdn_ref[...].astype(jnp.bfloat16),
            preferred_element_type=jnp.float32)

    @pl.when(e != 0)
    def _():
        out_ref[pl.ds(s * TS, TS), :] += contrib


@jax.jit
def _moe(x2, router_w, router_b, w_up, b_up, w_down, b_down):
    grid = (N_EXP, SEQ // TS)
    return pl.pallas_call(
        _moe_kernel,
        grid=grid,
        in_specs=[
            pl.BlockSpec((SEQ, D_MODEL), lambda e, s: (0, 0)),
            pl.BlockSpec((D_MODEL, N_EXP), lambda e, s: (0, 0)),
            pl.BlockSpec((N_EXP,), lambda e, s: (0,)),
            pl.BlockSpec((1, D_MODEL, EXP_DIM), lambda e, s: (e, 0, 0)),
            pl.BlockSpec((1, 1, EXP_DIM), lambda e, s: (e, 0, 0)),
            pl.BlockSpec((1, EXP_DIM, D_MODEL), lambda e, s: (e, 0, 0)),
            pl.BlockSpec((N_EXP, D_MODEL), lambda e, s: (0, 0)),
        ],
        out_specs=pl.BlockSpec((SEQ, D_MODEL), lambda e, s: (0, 0)),
        out_shape=jax.ShapeDtypeStruct((SEQ, D_MODEL), jnp.float32),
        scratch_shapes=[pltpu.VMEM((SEQ, N_EXP), jnp.float32)],
        compiler_params=pltpu.CompilerParams(
            dimension_semantics=("arbitrary", "arbitrary"),
        ),
    )(x2, router_w, router_b, w_up, b_up, w_down, b_down)


def kernel(x, router_w, router_b, w_up, b_up, w_down, b_down):
    b, seq, d = x.shape
    out = _moe(x.reshape(seq, d), router_w, router_b, w_up,
               b_up.reshape(N_EXP, 1, EXP_DIM), w_down, b_down)
    return out.reshape(b, seq, d)
